# baseline (speedup 1.0000x reference)
import jax
import jax.numpy as jnp
from jax.experimental import pallas as pl

N = 10000
HEADS = 4
HID = 64
D_OUT = 128


def _mm_body(x_ref, w_ref, o_ref):
    o_ref[...] = jnp.dot(x_ref[...], w_ref[...], preferred_element_type=jnp.float32)


def _matmul(x, w):
    return pl.pallas_call(
        _mm_body,
        out_shape=jax.ShapeDtypeStruct((x.shape[0], w.shape[1]), jnp.float32),
    )(x, w)


def _gat_layer(x, src, dst, W, a_src, a_dst, b, heads, ch):
    n = x.shape[0]
    h = _matmul(x, W).reshape(n, heads, ch)
    alpha_src = (h * a_src[None, :, :]).sum(-1)
    alpha_dst = (h * a_dst[None, :, :]).sum(-1)
    e = alpha_src[src] + alpha_dst[dst]
    e = jnp.where(e > 0, e, 0.2 * e)
    emax = jax.ops.segment_max(e, dst, num_segments=n)
    ex = jnp.exp(e - emax[dst])
    denom = jax.ops.segment_sum(ex, dst, num_segments=n)
    alpha = ex / (denom[dst] + 1e-16)
    out = jax.ops.segment_sum(h[src] * alpha[:, :, None], dst, num_segments=n)
    return out.reshape(n, heads * ch) + b


def _gat_branch(x, edge_index, p1, p2):
    n = x.shape[0]
    loop = jnp.arange(n, dtype=edge_index.dtype)
    src = jnp.concatenate([edge_index[0], loop])
    dst = jnp.concatenate([edge_index[1], loop])
    h = jax.nn.elu(_gat_layer(x, src, dst, p1[0], p1[1], p1[2], p1[3], HEADS, HID))
    h = jax.nn.elu(_gat_layer(h, src, dst, p2[0], p2[1], p2[2], p2[3], 1, D_OUT))
    return h


def kernel(x0, x1, edge_index0, edge_index1, W1_0, as1_0, ad1_0, b1_0, W2_0, as2_0, ad2_0, b2_0, W1_1, as1_1, ad1_1, b1_1, W2_1, as2_1, ad2_1, b2_1):
    out0 = _gat_branch(x0, edge_index0, (W1_0, as1_0, ad1_0, b1_0), (W2_0, as2_0, ad2_0, b2_0))
    out1 = _gat_branch(x1, edge_index1, (W1_1, as1_1, ad1_1, b1_1), (W2_1, as2_1, ad2_1, b2_1))
    return jnp.concatenate([out0, out1], axis=0)


# speedup vs baseline: 36.0189x; 36.0189x over previous
import functools

import jax
import jax.numpy as jnp
from jax import lax
from jax.experimental import pallas as pl
from jax.experimental.pallas import tpu as pltpu
from jax.experimental.pallas import tpu_sc as plsc

N = 10000
NP = 10240
E = 160000
EP = 172032
C = 4096
NB = 1024
F32 = jnp.float32

_mesh = plsc.VectorSubcoreMesh(core_axis_name="c", subcore_axis_name="s")
_CP_SC = pltpu.CompilerParams(needs_layout_passes=False)


def _tc_pre_body(x_ref, w_ref, a_ref, hT_ref, al_ref, gmax_ref):
    hT = lax.dot_general(w_ref[...], x_ref[...], (((0,), (1,)), ((), ())),
                         preferred_element_type=F32)
    hT_ref[...] = hT
    al = lax.dot_general(a_ref[...], hT, (((0,), (0,)), ((), ())),
                         preferred_element_type=F32)
    al_ref[...] = al
    rm = jnp.max(al, axis=1, keepdims=True)
    rmb = lax.broadcast_in_dim(rm, (8, 128), (0, 1))

    @pl.when(pl.program_id(0) == 0)
    def _():
        gmax_ref[...] = rmb

    @pl.when(pl.program_id(0) != 0)
    def _():
        gmax_ref[...] = jnp.maximum(gmax_ref[...], rmb)


def _tc_pre(xp, W, A, dh):
    din = xp.shape[1]
    return pl.pallas_call(
        _tc_pre_body,
        grid=(NP // NB,),
        in_specs=[
            pl.BlockSpec((NB, din), lambda i: (i, 0)),
            pl.BlockSpec((din, dh), lambda i: (0, 0)),
            pl.BlockSpec((dh, 8), lambda i: (0, 0)),
        ],
        out_specs=[
            pl.BlockSpec((dh, NB), lambda i: (0, i)),
            pl.BlockSpec((8, NB), lambda i: (0, i)),
            pl.BlockSpec((8, 128), lambda i: (0, 0)),
        ],
        out_shape=[
            jax.ShapeDtypeStruct((dh, NP), F32),
            jax.ShapeDtypeStruct((8, NP), F32),
            jax.ShapeDtypeStruct((8, 128), F32),
        ],
    )(xp, W, A)


def _tc_mid_body(acc_ref, den_ref, b_ref, w_ref, a_ref,
                 zT_ref, al_ref, gmax_ref):
    i = pl.program_id(0)
    acc = acc_ref[...]
    den = den_ref[...]
    col = lax.broadcasted_iota(jnp.int32, (1, NB), 1) + i * NB
    valid = col < N
    acc = jnp.where(lax.broadcast_in_dim(valid, (256, NB), (0, 1)), acc, 0.0)
    den = jnp.where(lax.broadcast_in_dim(valid, (4, NB), (0, 1)), den, 1.0)
    acc3 = acc.reshape(4, 64, NB)
    den3 = lax.broadcast_in_dim(den, (4, 64, NB), (0, 2))
    h = acc3 / (den3 + 1e-16) + b_ref[...].reshape(4, 64, 1)
    h = h.reshape(256, NB)
    h = jnp.where(h > 0, h, jnp.exp(h) - 1.0)
    z = lax.dot_general(w_ref[...], h, (((0,), (0,)), ((), ())),
                        preferred_element_type=F32)
    zT_ref[...] = z
    al2 = lax.dot_general(a_ref[...], z, (((0,), (0,)), ((), ())),
                          preferred_element_type=F32)
    al2p = jnp.concatenate([al2, jnp.full((6, NB), -1e30, F32)], axis=0)
    al_ref[...] = al2p
    rm = jnp.max(al2p, axis=1, keepdims=True)
    rmb = lax.broadcast_in_dim(rm, (8, 128), (0, 1))

    @pl.when(i == 0)
    def _():
        gmax_ref[...] = rmb

    @pl.when(i != 0)
    def _():
        gmax_ref[...] = jnp.maximum(gmax_ref[...], rmb)


def _tc_mid(accT, denT, b1c, W2, A2):
    return pl.pallas_call(
        _tc_mid_body,
        grid=(NP // NB,),
        in_specs=[
            pl.BlockSpec((256, NB), lambda i: (0, i)),
            pl.BlockSpec((4, NB), lambda i: (0, i)),
            pl.BlockSpec((256, 1), lambda i: (0, 0)),
            pl.BlockSpec((256, 128), lambda i: (0, 0)),
            pl.BlockSpec((128, 2), lambda i: (0, 0)),
        ],
        out_specs=[
            pl.BlockSpec((128, NB), lambda i: (0, i)),
            pl.BlockSpec((8, NB), lambda i: (0, i)),
            pl.BlockSpec((8, 128), lambda i: (0, 0)),
        ],
        out_shape=[
            jax.ShapeDtypeStruct((128, NP), F32),
            jax.ShapeDtypeStruct((8, NP), F32),
            jax.ShapeDtypeStruct((8, 128), F32),
        ],
    )(accT, denT, b1c, W2, A2)


def _tc_post_body(acc_ref, den_ref, b_ref, eye_ref, out_ref):
    acc = acc_ref[...]
    den = den_ref[...]
    h = acc / (lax.broadcast_in_dim(den, (128, NB), (0, 1)) + 1e-16)
    h = h + b_ref[...]
    h = jnp.where(h > 0, h, jnp.exp(h) - 1.0)
    out_ref[...] = lax.dot_general(h, eye_ref[...], (((0,), (0,)), ((), ())),
                                   preferred_element_type=F32)


def _tc_post(acc2T, den2, b2c, eye):
    return pl.pallas_call(
        _tc_post_body,
        grid=(NP // NB,),
        in_specs=[
            pl.BlockSpec((128, NB), lambda i: (0, i)),
            pl.BlockSpec((1, NB), lambda i: (0, i)),
            pl.BlockSpec((128, 1), lambda i: (0, 0)),
            pl.BlockSpec((128, 128), lambda i: (0, 0)),
        ],
        out_specs=pl.BlockSpec((NB, 128), lambda i: (i, 0)),
        out_shape=jax.ShapeDtypeStruct((NP, 128), F32),
    )(acc2T, den2, b2c, eye)


def _make_edge_kernel(heads, featc):
    nunits = featc // 4
    units_per_tec = nunits // 32
    dst_row = 4 if heads == 4 else 1
    chunks = EP // C

    @functools.partial(
        pl.kernel,
        out_type=(jax.ShapeDtypeStruct((featc * NP,), F32),
                  jax.ShapeDtypeStruct((8 * NP,), F32)),
        mesh=_mesh,
        compiler_params=_CP_SC,
        scratch_types=[
            pltpu.VMEM((4 * NP,), F32),
            pltpu.VMEM((4 * NP,), F32),
            pltpu.VMEM((NP,), F32),
            pltpu.VMEM((NP,), F32),
            pltpu.VMEM((NP,), F32),
            pltpu.VMEM((C,), jnp.int32),
            pltpu.VMEM((C,), jnp.int32),
            pltpu.VMEM((128,), F32),
            pltpu.VMEM((128,), F32),
        ],
    )
    def edge_kernel(tabT, alphaT, gmaxrow, src, dst, accT_o, den_o,
                    tab, acc, asr, ads, accd, sv, dv, gm1, gm2):
        cid = lax.axis_index("c")
        sid = lax.axis_index("s")
        wid = sid * 2 + cid
        for t in range(units_per_tec):
            u = wid * units_per_tec + t
            head = (u // 16) if heads == 4 else (u * 0)
            pltpu.sync_copy(tabT.at[pl.ds(u * (4 * NP), 4 * NP)], tab)
            pltpu.sync_copy(alphaT.at[pl.ds(head * NP, NP)], asr)
            pltpu.sync_copy(alphaT.at[pl.ds((dst_row + head) * NP, NP)], ads)
            pltpu.sync_copy(gmaxrow.at[pl.ds(head * 128, 128)], gm1)
            pltpu.sync_copy(gmaxrow.at[pl.ds((dst_row + head) * 128, 128)], gm2)
            b = gm1[pl.ds(0, 16)] + gm2[pl.ds(0, 16)]
            g = jnp.maximum(b, 0.2 * b)
            zeros = jnp.zeros((16,), F32)

            @plsc.parallel_loop(0, 4 * NP, 16, unroll=8)
            def _zero(o):
                acc[pl.ds(o, 16)] = zeros

            @plsc.parallel_loop(0, NP, 16, unroll=8)
            def _zerod(o):
                accd[pl.ds(o, 16)] = zeros

            def _chunk(ci, carry):
                pltpu.sync_copy(src.at[pl.ds(ci * C, C)], sv)
                pltpu.sync_copy(dst.at[pl.ds(ci * C, C)], dv)

                @plsc.parallel_loop(0, C, 16, unroll=4)
                def _body(o):
                    s = sv[pl.ds(o, 16)]
                    d = dv[pl.ds(o, 16)]
                    e = plsc.load_gather(asr, [s]) + plsc.load_gather(ads, [d])
                    e = jnp.maximum(e, 0.2 * e)
                    ex = jnp.exp(e - g)
                    for f in range(4):
                        tv = plsc.load_gather(tab, [s + jnp.int32(f * NP)])
                        plsc.addupdate_scatter(acc, [d + jnp.int32(f * NP)],
                                               tv * ex)
                    plsc.addupdate_scatter(accd, [d], ex)

                return carry

            lax.fori_loop(0, chunks, _chunk, 0)
            pltpu.sync_copy(acc, accT_o.at[pl.ds(u * (4 * NP), 4 * NP)])
            is_aug = (u % 16 == 0) if heads == 4 else (u == 0)

            @pl.when(is_aug)
            def _():
                pltpu.sync_copy(accd, den_o.at[pl.ds(head * NP, NP)])

    return edge_kernel


_edge_l1 = _make_edge_kernel(4, 256)
_edge_l2 = _make_edge_kernel(1, 128)


def _branch(x, edge_index, p1, p2):
    W1, as1, ad1, b1 = p1
    W2, as2, ad2, b2 = p2

    loop = jnp.arange(N, dtype=edge_index.dtype)
    src = jnp.concatenate([edge_index[0], loop])
    dst = jnp.concatenate([edge_index[1], loop])
    pad = jnp.full((EP - E - N,), NP - 1, dtype=src.dtype)
    src = jnp.concatenate([src, pad])
    dst = jnp.concatenate([dst, pad])

    xp = jnp.pad(x, ((0, NP - N), (0, 0)))

    eye4 = jnp.eye(4, dtype=F32)
    A1s = jnp.einsum("hc,hk->hck", as1, eye4).reshape(256, 4)
    A1d = jnp.einsum("hc,hk->hck", ad1, eye4).reshape(256, 4)
    A1 = jnp.concatenate([A1s, A1d], axis=1)
    A2 = jnp.stack([as2[0], ad2[0]], axis=1)

    h1T, alphaT, gmaxrow = _tc_pre(xp, W1, A1, 256)
    accT, denT = _edge_l1(h1T.reshape(-1), alphaT.reshape(-1),
                          gmaxrow.reshape(-1), src, dst)
    zT, alphaT2, gmax2row = _tc_mid(accT.reshape(256, NP),
                                    denT.reshape(8, NP)[:4],
                                    b1[:, None], W2, A2)
    acc2T, den2 = _edge_l2(zT.reshape(-1), alphaT2.reshape(-1),
                           gmax2row.reshape(-1), src, dst)
    outp = _tc_post(acc2T.reshape(128, NP), den2.reshape(8, NP)[:1],
                    b2[:, None], jnp.eye(128, dtype=F32))
    return outp[:N]


def kernel(x0, x1, edge_index0, edge_index1, W1_0, as1_0, ad1_0, b1_0, W2_0, as2_0, ad2_0, b2_0, W1_1, as1_1, ad1_1, b1_1, W2_1, as2_1, ad2_1, b2_1):
    out0 = _branch(x0, edge_index0, (W1_0, as1_0, ad1_0, b1_0), (W2_0, as2_0, ad2_0, b2_0))
    out1 = _branch(x1, edge_index1, (W1_1, as1_1, ad1_1, b1_1), (W2_1, as2_1, ad2_1, b2_1))
    return jnp.concatenate([out0, out1], axis=0)


# speedup vs baseline: 36.8477x; 1.0230x over previous
import functools

import jax
import jax.numpy as jnp
from jax import lax
from jax.experimental import pallas as pl
from jax.experimental.pallas import tpu as pltpu
from jax.experimental.pallas import tpu_sc as plsc

N = 10000
NP = 10240
E = 160000
EP = 172032
C = 4096
NB = 1024
F32 = jnp.float32

_mesh = plsc.VectorSubcoreMesh(core_axis_name="c", subcore_axis_name="s")
_CP_SC = pltpu.CompilerParams(needs_layout_passes=False)


def _tc_pre_body(x_ref, w_ref, a_ref, hT_ref, al_ref, gmax_ref):
    hT = lax.dot_general(w_ref[...], x_ref[...], (((0,), (1,)), ((), ())),
                         preferred_element_type=F32)
    hT_ref[...] = hT
    al = lax.dot_general(a_ref[...], hT, (((0,), (0,)), ((), ())),
                         preferred_element_type=F32)
    al_ref[...] = al
    rm = jnp.max(al, axis=1, keepdims=True)
    rmb = lax.broadcast_in_dim(rm, (8, 128), (0, 1))

    @pl.when(pl.program_id(0) == 0)
    def _():
        gmax_ref[...] = rmb

    @pl.when(pl.program_id(0) != 0)
    def _():
        gmax_ref[...] = jnp.maximum(gmax_ref[...], rmb)


def _tc_pre(xp, W, A, dh):
    din = xp.shape[1]
    return pl.pallas_call(
        _tc_pre_body,
        grid=(NP // NB,),
        in_specs=[
            pl.BlockSpec((NB, din), lambda i: (i, 0)),
            pl.BlockSpec((din, dh), lambda i: (0, 0)),
            pl.BlockSpec((dh, 8), lambda i: (0, 0)),
        ],
        out_specs=[
            pl.BlockSpec((dh, NB), lambda i: (0, i)),
            pl.BlockSpec((8, NB), lambda i: (0, i)),
            pl.BlockSpec((8, 128), lambda i: (0, 0)),
        ],
        out_shape=[
            jax.ShapeDtypeStruct((dh, NP), F32),
            jax.ShapeDtypeStruct((8, NP), F32),
            jax.ShapeDtypeStruct((8, 128), F32),
        ],
    )(xp, W, A)


def _tc_mid_body(acc_ref, den_ref, b_ref, w_ref, a_ref,
                 zT_ref, al_ref, gmax_ref):
    i = pl.program_id(0)
    acc = acc_ref[...]
    den = den_ref[...]
    col = lax.broadcasted_iota(jnp.int32, (1, NB), 1) + i * NB
    valid = col < N
    acc = jnp.where(lax.broadcast_in_dim(valid, (256, NB), (0, 1)), acc, 0.0)
    den = jnp.where(lax.broadcast_in_dim(valid, (4, NB), (0, 1)), den, 1.0)
    acc3 = acc.reshape(4, 64, NB)
    den3 = lax.broadcast_in_dim(den, (4, 64, NB), (0, 2))
    h = acc3 / (den3 + 1e-16) + b_ref[...].reshape(4, 64, 1)
    h = h.reshape(256, NB)
    h = jnp.where(h > 0, h, jnp.exp(h) - 1.0)
    z = lax.dot_general(w_ref[...], h, (((0,), (0,)), ((), ())),
                        preferred_element_type=F32)
    zT_ref[...] = z
    al2 = lax.dot_general(a_ref[...], z, (((0,), (0,)), ((), ())),
                          preferred_element_type=F32)
    al2p = jnp.concatenate([al2, jnp.full((6, NB), -1e30, F32)], axis=0)
    al_ref[...] = al2p
    rm = jnp.max(al2p, axis=1, keepdims=True)
    rmb = lax.broadcast_in_dim(rm, (8, 128), (0, 1))

    @pl.when(i == 0)
    def _():
        gmax_ref[...] = rmb

    @pl.when(i != 0)
    def _():
        gmax_ref[...] = jnp.maximum(gmax_ref[...], rmb)


def _tc_mid(accT, denT, b1c, W2, A2):
    return pl.pallas_call(
        _tc_mid_body,
        grid=(NP // NB,),
        in_specs=[
            pl.BlockSpec((256, NB), lambda i: (0, i)),
            pl.BlockSpec((4, NB), lambda i: (0, i)),
            pl.BlockSpec((256, 1), lambda i: (0, 0)),
            pl.BlockSpec((256, 128), lambda i: (0, 0)),
            pl.BlockSpec((128, 2), lambda i: (0, 0)),
        ],
        out_specs=[
            pl.BlockSpec((128, NB), lambda i: (0, i)),
            pl.BlockSpec((8, NB), lambda i: (0, i)),
            pl.BlockSpec((8, 128), lambda i: (0, 0)),
        ],
        out_shape=[
            jax.ShapeDtypeStruct((128, NP), F32),
            jax.ShapeDtypeStruct((8, NP), F32),
            jax.ShapeDtypeStruct((8, 128), F32),
        ],
    )(accT, denT, b1c, W2, A2)


def _tc_post_body(acc_ref, den_ref, b_ref, eye_ref, out_ref):
    acc = acc_ref[...]
    den = den_ref[...]
    h = acc / (lax.broadcast_in_dim(den, (128, NB), (0, 1)) + 1e-16)
    h = h + b_ref[...]
    h = jnp.where(h > 0, h, jnp.exp(h) - 1.0)
    out_ref[...] = lax.dot_general(h, eye_ref[...], (((0,), (0,)), ((), ())),
                                   preferred_element_type=F32)


def _tc_post(acc2T, den2, b2c, eye):
    return pl.pallas_call(
        _tc_post_body,
        grid=(NP // NB,),
        in_specs=[
            pl.BlockSpec((128, NB), lambda i: (0, i)),
            pl.BlockSpec((1, NB), lambda i: (0, i)),
            pl.BlockSpec((128, 1), lambda i: (0, 0)),
            pl.BlockSpec((128, 128), lambda i: (0, 0)),
        ],
        out_specs=pl.BlockSpec((NB, 128), lambda i: (i, 0)),
        out_shape=jax.ShapeDtypeStruct((NP, 128), F32),
    )(acc2T, den2, b2c, eye)


def _make_edge_kernel(heads, featc):
    nunits = featc // 4
    units_per_tec = nunits // 32
    dst_row = 4 if heads == 4 else 1
    chunks = EP // C

    @functools.partial(
        pl.kernel,
        out_type=(jax.ShapeDtypeStruct((featc * NP,), F32),
                  jax.ShapeDtypeStruct((8 * NP,), F32)),
        mesh=_mesh,
        compiler_params=_CP_SC,
        scratch_types=(
            [pltpu.VMEM((NP,), F32) for _ in range(4)]
            + [pltpu.VMEM((NP,), F32) for _ in range(4)]
            + [
                pltpu.VMEM((NP,), F32),
                pltpu.VMEM((NP,), F32),
                pltpu.VMEM((NP,), F32),
                pltpu.VMEM((C,), jnp.int32),
                pltpu.VMEM((C,), jnp.int32),
                pltpu.VMEM((128,), F32),
                pltpu.VMEM((128,), F32),
            ]
        ),
    )
    def edge_kernel(tabT, alphaT, gmaxrow, src, dst, accT_o, den_o,
                    t0, t1, t2, t3, a0, a1, a2, a3,
                    asr, ads, accd, sv, dv, gm1, gm2):
        tabs = (t0, t1, t2, t3)
        accs = (a0, a1, a2, a3)
        cid = lax.axis_index("c")
        sid = lax.axis_index("s")
        wid = sid * 2 + cid
        for t in range(units_per_tec):
            u = wid * units_per_tec + t
            head = (u // 16) if heads == 4 else (u * 0)
            for f in range(4):
                pltpu.sync_copy(tabT.at[pl.ds((u * 4 + f) * NP, NP)], tabs[f])
            pltpu.sync_copy(alphaT.at[pl.ds(head * NP, NP)], asr)
            pltpu.sync_copy(alphaT.at[pl.ds((dst_row + head) * NP, NP)], ads)
            pltpu.sync_copy(gmaxrow.at[pl.ds(head * 128, 128)], gm1)
            pltpu.sync_copy(gmaxrow.at[pl.ds((dst_row + head) * 128, 128)], gm2)
            b = gm1[pl.ds(0, 16)] + gm2[pl.ds(0, 16)]
            g = jnp.maximum(b, 0.2 * b)
            zeros = jnp.zeros((16,), F32)

            @plsc.parallel_loop(0, NP, 16, unroll=8)
            def _zero(o):
                for f in range(4):
                    accs[f][pl.ds(o, 16)] = zeros
                accd[pl.ds(o, 16)] = zeros

            def _chunk(ci, carry):
                pltpu.sync_copy(src.at[pl.ds(ci * C, C)], sv)
                pltpu.sync_copy(dst.at[pl.ds(ci * C, C)], dv)

                @plsc.parallel_loop(0, C, 16, unroll=4)
                def _body(o):
                    s = sv[pl.ds(o, 16)]
                    d = dv[pl.ds(o, 16)]
                    e = plsc.load_gather(asr, [s]) + plsc.load_gather(ads, [d])
                    e = jnp.maximum(e, 0.2 * e)
                    ex = jnp.exp(e - g)
                    for f in range(4):
                        tv = plsc.load_gather(tabs[f], [s])
                        plsc.addupdate_scatter(accs[f], [d], tv * ex)
                    plsc.addupdate_scatter(accd, [d], ex)

                return carry

            lax.fori_loop(0, chunks, _chunk, 0)
            for f in range(4):
                pltpu.sync_copy(accs[f], accT_o.at[pl.ds((u * 4 + f) * NP, NP)])
            is_aug = (u % 16 == 0) if heads == 4 else (u == 0)

            @pl.when(is_aug)
            def _():
                pltpu.sync_copy(accd, den_o.at[pl.ds(head * NP, NP)])

    return edge_kernel


_edge_l1 = _make_edge_kernel(4, 256)
_edge_l2 = _make_edge_kernel(1, 128)


def _branch(x, edge_index, p1, p2):
    W1, as1, ad1, b1 = p1
    W2, as2, ad2, b2 = p2

    loop = jnp.arange(N, dtype=edge_index.dtype)
    src = jnp.concatenate([edge_index[0], loop])
    dst = jnp.concatenate([edge_index[1], loop])
    pad = jnp.full((EP - E - N,), NP - 1, dtype=src.dtype)
    src = jnp.concatenate([src, pad])
    dst = jnp.concatenate([dst, pad])

    xp = jnp.pad(x, ((0, NP - N), (0, 0)))

    eye4 = jnp.eye(4, dtype=F32)
    A1s = jnp.einsum("hc,hk->hck", as1, eye4).reshape(256, 4)
    A1d = jnp.einsum("hc,hk->hck", ad1, eye4).reshape(256, 4)
    A1 = jnp.concatenate([A1s, A1d], axis=1)
    A2 = jnp.stack([as2[0], ad2[0]], axis=1)

    h1T, alphaT, gmaxrow = _tc_pre(xp, W1, A1, 256)
    accT, denT = _edge_l1(h1T.reshape(-1), alphaT.reshape(-1),
                          gmaxrow.reshape(-1), src, dst)
    zT, alphaT2, gmax2row = _tc_mid(accT.reshape(256, NP),
                                    denT.reshape(8, NP)[:4],
                                    b1[:, None], W2, A2)
    acc2T, den2 = _edge_l2(zT.reshape(-1), alphaT2.reshape(-1),
                           gmax2row.reshape(-1), src, dst)
    outp = _tc_post(acc2T.reshape(128, NP), den2.reshape(8, NP)[:1],
                    b2[:, None], jnp.eye(128, dtype=F32))
    return outp[:N]


def kernel(x0, x1, edge_index0, edge_index1, W1_0, as1_0, ad1_0, b1_0, W2_0, as2_0, ad2_0, b2_0, W1_1, as1_1, ad1_1, b1_1, W2_1, as2_1, ad2_1, b2_1):
    out0 = _branch(x0, edge_index0, (W1_0, as1_0, ad1_0, b1_0), (W2_0, as2_0, ad2_0, b2_0))
    out1 = _branch(x1, edge_index1, (W1_1, as1_1, ad1_1, b1_1), (W2_1, as2_1, ad2_1, b2_1))
    return jnp.concatenate([out0, out1], axis=0)


# speedup vs baseline: 50.7133x; 1.3763x over previous
import functools

import jax
import jax.numpy as jnp
from jax import lax
from jax.experimental import pallas as pl
from jax.experimental.pallas import tpu as pltpu
from jax.experimental.pallas import tpu_sc as plsc

N = 10000
NP = 10240
E = 160000
EP = 172032
C = 4096
NB = 1024
F32 = jnp.float32

_mesh = plsc.VectorSubcoreMesh(core_axis_name="c", subcore_axis_name="s")
_CP_SC = pltpu.CompilerParams(needs_layout_passes=False)


def _tc_pre_body(x_ref, w_ref, a_ref, hT_ref, al_ref, gmax_ref):
    hT = lax.dot_general(w_ref[...], x_ref[...], (((0,), (1,)), ((), ())),
                         preferred_element_type=F32)
    hT_ref[...] = hT
    al = lax.dot_general(a_ref[...], hT, (((0,), (0,)), ((), ())),
                         preferred_element_type=F32)
    al_ref[...] = al
    rm = jnp.max(al, axis=1, keepdims=True)
    rmb = lax.broadcast_in_dim(rm, (8, 128), (0, 1))

    @pl.when(pl.program_id(0) == 0)
    def _():
        gmax_ref[...] = rmb

    @pl.when(pl.program_id(0) != 0)
    def _():
        gmax_ref[...] = jnp.maximum(gmax_ref[...], rmb)


def _tc_pre(xp, W, A, dh):
    din = xp.shape[1]
    return pl.pallas_call(
        _tc_pre_body,
        grid=(NP // NB,),
        in_specs=[
            pl.BlockSpec((NB, din), lambda i: (i, 0)),
            pl.BlockSpec((din, dh), lambda i: (0, 0)),
            pl.BlockSpec((dh, 8), lambda i: (0, 0)),
        ],
        out_specs=[
            pl.BlockSpec((dh, NB), lambda i: (0, i)),
            pl.BlockSpec((8, NB), lambda i: (0, i)),
            pl.BlockSpec((8, 128), lambda i: (0, 0)),
        ],
        out_shape=[
            jax.ShapeDtypeStruct((dh, NP), F32),
            jax.ShapeDtypeStruct((8, NP), F32),
            jax.ShapeDtypeStruct((8, 128), F32),
        ],
    )(xp, W, A)


def _tc_mid_body(acc_ref, den_ref, b_ref, w_ref, a_ref,
                 zT_ref, al_ref, gmax_ref):
    i = pl.program_id(0)
    acc = acc_ref[...]
    den = den_ref[...]
    col = lax.broadcasted_iota(jnp.int32, (1, NB), 1) + i * NB
    valid = col < N
    acc = jnp.where(lax.broadcast_in_dim(valid, (256, NB), (0, 1)), acc, 0.0)
    den = jnp.where(lax.broadcast_in_dim(valid, (4, NB), (0, 1)), den, 1.0)
    acc3 = acc.reshape(4, 64, NB)
    den3 = lax.broadcast_in_dim(den, (4, 64, NB), (0, 2))
    h = acc3 / (den3 + 1e-16) + b_ref[...].reshape(4, 64, 1)
    h = h.reshape(256, NB)
    h = jnp.where(h > 0, h, jnp.exp(h) - 1.0)
    z = lax.dot_general(w_ref[...], h, (((0,), (0,)), ((), ())),
                        preferred_element_type=F32)
    zT_ref[...] = z
    al2 = lax.dot_general(a_ref[...], z, (((0,), (0,)), ((), ())),
                          preferred_element_type=F32)
    al2p = jnp.concatenate([al2, jnp.full((6, NB), -1e30, F32)], axis=0)
    al_ref[...] = al2p
    rm = jnp.max(al2p, axis=1, keepdims=True)
    rmb = lax.broadcast_in_dim(rm, (8, 128), (0, 1))

    @pl.when(i == 0)
    def _():
        gmax_ref[...] = rmb

    @pl.when(i != 0)
    def _():
        gmax_ref[...] = jnp.maximum(gmax_ref[...], rmb)


def _tc_mid(accT, denT, b1c, W2, A2):
    return pl.pallas_call(
        _tc_mid_body,
        grid=(NP // NB,),
        in_specs=[
            pl.BlockSpec((256, NB), lambda i: (0, i)),
            pl.BlockSpec((4, NB), lambda i: (0, i)),
            pl.BlockSpec((256, 1), lambda i: (0, 0)),
            pl.BlockSpec((256, 128), lambda i: (0, 0)),
            pl.BlockSpec((128, 2), lambda i: (0, 0)),
        ],
        out_specs=[
            pl.BlockSpec((128, NB), lambda i: (0, i)),
            pl.BlockSpec((8, NB), lambda i: (0, i)),
            pl.BlockSpec((8, 128), lambda i: (0, 0)),
        ],
        out_shape=[
            jax.ShapeDtypeStruct((128, NP), F32),
            jax.ShapeDtypeStruct((8, NP), F32),
            jax.ShapeDtypeStruct((8, 128), F32),
        ],
    )(accT, denT, b1c, W2, A2)


def _tc_post_body(acc_ref, den_ref, b_ref, eye_ref, out_ref):
    acc = acc_ref[...]
    den = den_ref[...]
    h = acc / (lax.broadcast_in_dim(den, (128, NB), (0, 1)) + 1e-16)
    h = h + b_ref[...]
    h = jnp.where(h > 0, h, jnp.exp(h) - 1.0)
    out_ref[...] = lax.dot_general(h, eye_ref[...], (((0,), (0,)), ((), ())),
                                   preferred_element_type=F32)


def _tc_post(acc2T, den2, b2c, eye):
    return pl.pallas_call(
        _tc_post_body,
        grid=(NP // NB,),
        in_specs=[
            pl.BlockSpec((128, NB), lambda i: (0, i)),
            pl.BlockSpec((1, NB), lambda i: (0, i)),
            pl.BlockSpec((128, 1), lambda i: (0, 0)),
            pl.BlockSpec((128, 128), lambda i: (0, 0)),
        ],
        out_specs=pl.BlockSpec((NB, 128), lambda i: (i, 0)),
        out_shape=jax.ShapeDtypeStruct((NP, 128), F32),
    )(acc2T, den2, b2c, eye)


def _make_edge_kernel(heads, featc):
    nunits = featc // 4
    units_per_tec = nunits // 32
    dst_row = 4 if heads == 4 else 1
    chunks = EP // C

    @functools.partial(
        pl.kernel,
        out_type=(jax.ShapeDtypeStruct((featc * NP,), F32),
                  jax.ShapeDtypeStruct((8 * NP,), F32)),
        mesh=_mesh,
        compiler_params=_CP_SC,
        scratch_types=(
            [pltpu.VMEM((NP,), F32) for _ in range(4)]
            + [pltpu.VMEM((NP,), F32) for _ in range(4)]
            + [
                pltpu.VMEM((NP,), F32),
                pltpu.VMEM((NP,), F32),
                pltpu.VMEM((NP,), F32),
                pltpu.VMEM((C,), jnp.int32),
                pltpu.VMEM((C,), jnp.int32),
                pltpu.VMEM((C,), jnp.int32),
                pltpu.VMEM((C,), jnp.int32),
                pltpu.VMEM((128,), F32),
                pltpu.VMEM((128,), F32),
                pltpu.SemaphoreType.DMA,
                pltpu.SemaphoreType.DMA,
                pltpu.SemaphoreType.DMA,
                pltpu.SemaphoreType.DMA,
            ]
        ),
    )
    def edge_kernel(tabT, alphaT, gmaxrow, src, dst, accT_o, den_o,
                    t0, t1, t2, t3, a0, a1, a2, a3,
                    asr, ads, accd, sv0, dv0, sv1, dv1, gm1, gm2,
                    ss0, sd0, ss1, sd1):
        tabs = (t0, t1, t2, t3)
        accs = (a0, a1, a2, a3)
        cid = lax.axis_index("c")
        sid = lax.axis_index("s")
        wid = sid * 2 + cid
        for t in range(units_per_tec):
            u = wid * units_per_tec + t
            head = (u // 16) if heads == 4 else (u * 0)
            for f in range(4):
                pltpu.sync_copy(tabT.at[pl.ds((u * 4 + f) * NP, NP)], tabs[f])
            pltpu.sync_copy(alphaT.at[pl.ds(head * NP, NP)], asr)
            pltpu.sync_copy(alphaT.at[pl.ds((dst_row + head) * NP, NP)], ads)
            pltpu.sync_copy(gmaxrow.at[pl.ds(head * 128, 128)], gm1)
            pltpu.sync_copy(gmaxrow.at[pl.ds((dst_row + head) * 128, 128)], gm2)
            b = gm1[pl.ds(0, 16)] + gm2[pl.ds(0, 16)]
            g = jnp.maximum(b, 0.2 * b)
            zeros = jnp.zeros((16,), F32)

            @plsc.parallel_loop(0, NP, 16, unroll=8)
            def _zero(o):
                for f in range(4):
                    accs[f][pl.ds(o, 16)] = zeros
                accd[pl.ds(o, 16)] = zeros

            def _start(ci, svb, dvb, sems):
                pltpu.async_copy(src.at[pl.ds(ci * C, C)], svb, sems[0])
                pltpu.async_copy(dst.at[pl.ds(ci * C, C)], dvb, sems[1])

            def _wait(svb, dvb, sems):
                pltpu.make_async_copy(src.at[pl.ds(0, C)], svb, sems[0]).wait()
                pltpu.make_async_copy(dst.at[pl.ds(0, C)], dvb, sems[1]).wait()

            def _run(svb, dvb):
                @plsc.parallel_loop(0, C, 16, unroll=4)
                def _body(o):
                    s = svb[pl.ds(o, 16)]
                    d = dvb[pl.ds(o, 16)]
                    e = plsc.load_gather(asr, [s]) + plsc.load_gather(ads, [d])
                    e = jnp.maximum(e, 0.2 * e)
                    ex = jnp.exp(e - g)
                    for f in range(4):
                        tv = plsc.load_gather(tabs[f], [s])
                        plsc.addupdate_scatter(accs[f], [d], tv * ex)
                    plsc.addupdate_scatter(accd, [d], ex)

            _start(0, sv0, dv0, (ss0, sd0))

            def _pair(j, carry):
                ci = 2 * j
                _start(ci + 1, sv1, dv1, (ss1, sd1))
                _wait(sv0, dv0, (ss0, sd0))
                _run(sv0, dv0)
                _start(jnp.minimum(ci + 2, chunks - 1), sv0, dv0, (ss0, sd0))
                _wait(sv1, dv1, (ss1, sd1))
                _run(sv1, dv1)
                return carry

            lax.fori_loop(0, chunks // 2, _pair, 0)
            _wait(sv0, dv0, (ss0, sd0))
            for f in range(4):
                pltpu.sync_copy(accs[f], accT_o.at[pl.ds((u * 4 + f) * NP, NP)])
            is_aug = (u % 16 == 0) if heads == 4 else (u == 0)

            @pl.when(is_aug)
            def _():
                pltpu.sync_copy(accd, den_o.at[pl.ds(head * NP, NP)])

    return edge_kernel


_edge_l1 = _make_edge_kernel(4, 256)
_edge_l2 = _make_edge_kernel(1, 128)


def _branch(x, edge_index, p1, p2):
    W1, as1, ad1, b1 = p1
    W2, as2, ad2, b2 = p2

    loop = jnp.arange(N, dtype=edge_index.dtype)
    src = jnp.concatenate([edge_index[0], loop])
    dst = jnp.concatenate([edge_index[1], loop])
    pad = jnp.full((EP - E - N,), NP - 1, dtype=src.dtype)
    src = jnp.concatenate([src, pad])
    dst = jnp.concatenate([dst, pad])

    xp = jnp.pad(x, ((0, NP - N), (0, 0)))

    eye4 = jnp.eye(4, dtype=F32)
    A1s = jnp.einsum("hc,hk->hck", as1, eye4).reshape(256, 4)
    A1d = jnp.einsum("hc,hk->hck", ad1, eye4).reshape(256, 4)
    A1 = jnp.concatenate([A1s, A1d], axis=1)
    A2 = jnp.stack([as2[0], ad2[0]], axis=1)

    h1T, alphaT, gmaxrow = _tc_pre(xp, W1, A1, 256)
    accT, denT = _edge_l1(h1T.reshape(-1), alphaT.reshape(-1),
                          gmaxrow.reshape(-1), src, dst)
    zT, alphaT2, gmax2row = _tc_mid(accT.reshape(256, NP),
                                    denT.reshape(8, NP)[:4],
                                    b1[:, None], W2, A2)
    acc2T, den2 = _edge_l2(zT.reshape(-1), alphaT2.reshape(-1),
                           gmax2row.reshape(-1), src, dst)
    outp = _tc_post(acc2T.reshape(128, NP), den2.reshape(8, NP)[:1],
                    b2[:, None], jnp.eye(128, dtype=F32))
    return outp[:N]


def kernel(x0, x1, edge_index0, edge_index1, W1_0, as1_0, ad1_0, b1_0, W2_0, as2_0, ad2_0, b2_0, W1_1, as1_1, ad1_1, b1_1, W2_1, as2_1, ad2_1, b2_1):
    out0 = _branch(x0, edge_index0, (W1_0, as1_0, ad1_0, b1_0), (W2_0, as2_0, ad2_0, b2_0))
    out1 = _branch(x1, edge_index1, (W1_1, as1_1, ad1_1, b1_1), (W2_1, as2_1, ad2_1, b2_1))
    return jnp.concatenate([out0, out1], axis=0)
